# Initial kernel scaffold; baseline (speedup 1.0000x reference)
import jax
import jax.numpy as jnp
from jax.experimental import pallas as pl


def kernel(x, edge_index, batch, target, W1, b1, W2, b2, lin_W, lin_b):
    raise NotImplementedError("write your pallas kernel here")


# speedup vs baseline: 13.6389x; 13.6389x over previous
import functools
import jax
import jax.numpy as jnp
from jax import lax
from jax.experimental import pallas as pl
from jax.experimental.pallas import tpu as pltpu
from jax.experimental.pallas import tpu_sc as plsc

N = 10000
D = 128
E = 320000
B = 100
NC = 2
NS = 16
NW = NC * NS
L = 16
NP = 10240
EPT = E // NW
K = 80
NCHUNK = EPT // K

@functools.lru_cache(maxsize=1)
def _mesh():
  return plsc.VectorSubcoreMesh(core_axis_name="c", subcore_axis_name="s",
                                num_cores=NC, num_subcores=NS)


def _f32(shape):
  return jax.ShapeDtypeStruct(shape, jnp.float32)


def _i32(shape):
  return jax.ShapeDtypeStruct(shape, jnp.int32)


def _hist_vreg(vals, ref, scr16):
  iota = lax.iota(jnp.int32, 16)
  s, _ = plsc.sort_key_val(vals, vals)
  scr16[...] = s
  prev = plsc.load_gather(scr16, [jnp.maximum(iota - 1, 0)])
  nxt = plsc.load_gather(scr16, [jnp.minimum(iota + 1, 15)])
  run_start = (iota == 0) | (s != prev)
  run_end = (iota == 15) | (s != nxt)
  rsi = plsc.cummax(jnp.where(run_start, iota, 0))
  cnt = (iota - rsi + 1).astype(jnp.float32)
  plsc.addupdate_scatter(ref, [s // 16, s % 16], cnt, mask=run_end)


def _hist_kernel(dst_hbm, batchp_hbm, deg_out, cnt_out,
                 deg_v, cnt_v, chunk_v, bchunk_v, scr16):
  cid = lax.axis_index("c")
  sid = lax.axis_index("s")
  wid = sid * NC + cid

  def z_deg(i, _):
    deg_v[i, :] = jnp.zeros((16,), jnp.float32)
    return 0
  lax.fori_loop(0, 640, z_deg, 0)
  for i in range(16):
    cnt_v[i, :] = jnp.zeros((16,), jnp.float32)

  def chunk_body(c, _):
    pltpu.sync_copy(dst_hbm.at[pl.ds(wid * EPT + c * 400, 400)], chunk_v)
    def vreg_body(j, _):
      _hist_vreg(chunk_v[pl.ds(j * 16, 16)], deg_v, scr16)
      return 0
    lax.fori_loop(0, 25, vreg_body, 0)
    return 0
  lax.fori_loop(0, 25, chunk_body, 0)

  pltpu.sync_copy(batchp_hbm.at[pl.ds(wid * 320, 320)], bchunk_v)
  def bvreg_body(j, _):
    _hist_vreg(bchunk_v[pl.ds(j * 16, 16)], cnt_v, scr16)
    return 0
  lax.fori_loop(0, 20, bvreg_body, 0)

  pltpu.sync_copy(deg_v, deg_out.at[wid])
  pltpu.sync_copy(cnt_v, cnt_out.at[wid])


@functools.lru_cache(maxsize=1)
def _hist():
  @functools.partial(
      pl.kernel,
      out_type=[_f32((NW, 640, 16)), _f32((NW, 16, 16))],
      mesh=_mesh(),
      scratch_types=[
          pltpu.VMEM((640, 16), jnp.float32),
          pltpu.VMEM((16, 16), jnp.float32),
          pltpu.VMEM((400,), jnp.int32),
          pltpu.VMEM((320,), jnp.int32),
          pltpu.VMEM((16,), jnp.int32),
      ],
      compiler_params=pltpu.CompilerParams(needs_layout_passes=False),
  )
  def hist(dst_hbm, batchp_hbm, deg_out, cnt_out, *scratch):
    _hist_kernel(dst_hbm, batchp_hbm, deg_out, cnt_out, *scratch)
  return hist


def _agg_kernel(h_hbm, src_hbm, dst_hbm, p_out,
                rows, sidx, didx, acc):
  cid = lax.axis_index("c")
  sid = lax.axis_index("s")
  wid = sid * NC + cid

  def zrow(i, _):
    for j in range(8):
      rows[i, pl.ds(j * 16, 16)] = jnp.zeros((16,), jnp.float32)
    return 0
  lax.fori_loop(0, K, zrow, 0)
  for k in range(8):
    pltpu.sync_copy(rows, acc.at[pl.ds(sid * 640 + k * K, K)])
  plsc.subcore_barrier()

  def chunk(c, _):
    base = wid * EPT + c * K
    pltpu.sync_copy(src_hbm.at[pl.ds(base, K)], sidx)
    pltpu.sync_copy(dst_hbm.at[pl.ds(base, K)], didx)
    pltpu.sync_copy(h_hbm.at[sidx], rows)
    pltpu.sync_copy(rows, acc.at[didx], add=True)
    return 0
  lax.fori_loop(0, NCHUNK, chunk, 0)
  plsc.subcore_barrier()

  for k in range(8):
    r0 = sid * 640 + k * K
    pltpu.sync_copy(acc.at[pl.ds(r0, K)], p_out.at[cid, pl.ds(r0, K)])


@functools.lru_cache(maxsize=1)
def _agg():
  @functools.partial(
      pl.kernel,
      out_type=_f32((NC, NP, D)),
      mesh=_mesh(),
      scratch_types=[
          pltpu.VMEM((K, D), jnp.float32),
          pltpu.VMEM((K,), jnp.int32),
          pltpu.VMEM((K,), jnp.int32),
          pltpu.VMEM_SHARED((NP, D), jnp.float32),
      ],
      compiler_params=pltpu.CompilerParams(needs_layout_passes=False),
  )
  def agg(h_hbm, src_hbm, dst_hbm, p_out, *scratch):
    _agg_kernel(h_hbm, src_hbm, dst_hbm, p_out, *scratch)
  return agg


def _qgather_kernel(qf_hbm, hs2_hbm, dinv_hbm, gq_hbm, gqb_hbm, b2_hbm,
                    qe_out, dinv_v, b2_v, g8, g8b, dscr,
                    q0v, q1v, hsv, outv):
  cid = lax.axis_index("c")
  sid = lax.axis_index("s")
  wid = sid * NC + cid

  @pl.when(wid < 25)
  def _():
    pltpu.sync_copy(dinv_hbm, dinv_v)
    pltpu.sync_copy(b2_hbm, b2_v)
    pltpu.sync_copy(gq_hbm.at[pl.ds(wid * 8, 8)], g8)
    pltpu.sync_copy(gqb_hbm.at[pl.ds(wid * 8, 8)], g8b)
    iota = lax.iota(jnp.int32, 16)
    g16 = plsc.load_gather(g8, [jnp.minimum(iota, 7)])
    dv16 = plsc.load_gather(dinv_v, [g16])
    dscr[...] = dv16
    pltpu.sync_copy(qf_hbm.at[g8], q0v)
    pltpu.sync_copy(qf_hbm.at[g8b], q1v)
    pltpu.sync_copy(hs2_hbm.at[g8], hsv)
    for r in range(8):
      dr = dv16[r]
      for c in range(8):
        sl = pl.ds(c * 16, 16)
        vec = q0v[r, sl] + q1v[r, sl] + hsv[r, sl]
        outv[r, sl] = dr * vec + b2_v[sl]
    pltpu.sync_copy(outv, qe_out.at[pl.ds(wid * 8, 8)])


@functools.lru_cache(maxsize=1)
def _qgather():
  @functools.partial(
      pl.kernel,
      out_type=_f32((200, D)),
      mesh=_mesh(),
      scratch_types=[
          pltpu.VMEM((NP,), jnp.float32),
          pltpu.VMEM((D,), jnp.float32),
          pltpu.VMEM((8,), jnp.int32),
          pltpu.VMEM((8,), jnp.int32),
          pltpu.VMEM((16,), jnp.float32),
          pltpu.VMEM((8, D), jnp.float32),
          pltpu.VMEM((8, D), jnp.float32),
          pltpu.VMEM((8, D), jnp.float32),
          pltpu.VMEM((8, D), jnp.float32),
      ],
      compiler_params=pltpu.CompilerParams(needs_layout_passes=False),
  )
  def qgather(qf, hs2, dinv, gq, gqb, b2, qe_out, *scratch):
    _qgather_kernel(qf, hs2, dinv, gq, gqb, b2, qe_out, *scratch)
  return qgather


def _t0_body(deg_ref, cnt_ref, targ_ref, dinv_ref, gidx_ref):
  deg = jnp.sum(deg_ref[...], axis=0, keepdims=True)
  dinv_ref[...] = lax.rsqrt(1.0 + deg)
  counts = jnp.sum(cnt_ref[...], axis=0).reshape(1, 256)
  ri = lax.broadcasted_iota(jnp.int32, (256, 128), 0)
  ci = lax.broadcasted_iota(jnp.int32, (256, 128), 1)
  ut = ((ri < ci) & (ri < B)).astype(jnp.float32)
  starts = jnp.dot(counts, ut, preferred_element_type=jnp.float32)
  g = jnp.clip(starts.astype(jnp.int32) + targ_ref[...], 0, N - 1)
  gidx_ref[...] = jnp.concatenate([g, g + NP], axis=0)


def _t0(deg_flat, cnt2, targ2):
  return pl.pallas_call(
      _t0_body,
      out_shape=[_f32((1, NP)), _i32((4, 128))],
  )(deg_flat, cnt2, targ2)


RB = 2000


def _t1_body(x_ref, w_ref, dinv_ref, hs_ref):
  h1p = jnp.dot(x_ref[...], w_ref[...], preferred_element_type=jnp.float32)
  hs_ref[...] = h1p * dinv_ref[...]


def _t1(x, w1, dinv2d):
  return pl.pallas_call(
      _t1_body,
      grid=(N // RB,),
      in_specs=[
          pl.BlockSpec((RB, D), lambda i: (i, 0)),
          pl.BlockSpec((D, D), lambda i: (0, 0)),
          pl.BlockSpec((RB, 1), lambda i: (i, 0)),
      ],
      out_specs=pl.BlockSpec((RB, D), lambda i: (i, 0)),
      out_shape=_f32((N, D)),
  )(x, w1, dinv2d)


def _t2_body(p_ref, hs1_ref, dinv_ref, b1_ref, w2_ref, hs2_ref):
  dinv = dinv_ref[...]
  p = p_ref[...]
  h1 = dinv * (p[0] + p[1] + hs1_ref[...]) + b1_ref[...]
  h1 = jnp.maximum(h1, 0.0)
  h2p = jnp.dot(h1, w2_ref[...], preferred_element_type=jnp.float32)
  hs2_ref[...] = dinv * h2p


def _t2(p, hs1, dinv, b1, w2):
  return pl.pallas_call(
      _t2_body,
      grid=(N // RB,),
      in_specs=[
          pl.BlockSpec((NC, RB, D), lambda i: (0, i, 0)),
          pl.BlockSpec((RB, D), lambda i: (i, 0)),
          pl.BlockSpec((RB, 1), lambda i: (i, 0)),
          pl.BlockSpec((1, D), lambda i: (0, 0)),
          pl.BlockSpec((D, D), lambda i: (0, 0)),
      ],
      out_specs=pl.BlockSpec((RB, D), lambda i: (i, 0)),
      out_shape=_f32((N, D)),
  )(p, hs1, dinv, b1, w2)


def _t4_body(qe_ref, w_ref, b_ref, out_ref):
  out_ref[...] = jnp.dot(qe_ref[...], w_ref[...],
                         preferred_element_type=jnp.float32) + b_ref[...]


def _t4(qe2, w, b):
  return pl.pallas_call(
      _t4_body,
      out_shape=_f32((B, 128)),
  )(qe2, w, b)


def kernel(x, edge_index, batch, target, W1, b1, W2, b2, lin_W, lin_b):
  src = edge_index[0]
  dst = edge_index[1]
  batchp = jnp.concatenate(
      [batch, jnp.full((NW * 320 - N,), B, dtype=jnp.int32)])
  targ2 = jnp.zeros((NC, 128), jnp.int32)
  targ2 = targ2.at[0, :B].set(target[:, 0]).at[1, :B].set(target[:, 1])

  deg_q, cnt_q = _hist()(dst, batchp)
  deg_flat = deg_q.reshape(NW, NP)
  cnt2 = cnt_q.reshape(NW, 256)

  dinv_flat, gidx = _t0(deg_flat, cnt2, targ2)
  dinv2d = dinv_flat.reshape(NP, 1)
  hs1 = _t1(x, W1, dinv2d)
  p = _agg()(hs1, src, dst)
  hs2 = _t2(p, hs1, dinv2d, b1.reshape(1, D), W2)
  q = _agg()(hs2, src, dst)

  gq = jnp.stack([gidx[0, :B], gidx[1, :B]], axis=1).reshape(2 * B)
  gqb = jnp.stack([gidx[2, :B], gidx[3, :B]], axis=1).reshape(2 * B)
  qe = _qgather()(q.reshape(NC * NP, D), hs2, dinv_flat.reshape(NP), gq, gqb, b2)
  qe2 = qe.reshape(B, 2 * D)

  lin_Wp = jnp.zeros((2 * D, 128), jnp.float32).at[:, :18].set(lin_W)
  lin_bp = jnp.zeros((1, 128), jnp.float32).at[0, :18].set(lin_b)
  logits = _t4(qe2, lin_Wp, lin_bp)
  return logits[:, :18]


# speedup vs baseline: 24.4115x; 1.7898x over previous
import functools
import jax
import jax.numpy as jnp
from jax import lax
from jax.experimental import pallas as pl
from jax.experimental.pallas import tpu as pltpu
from jax.experimental.pallas import tpu_sc as plsc

N = 10000
D = 128
E = 320000
B = 100
NC = 2
NS = 16
NW = NC * NS
L = 16
NP = 10240
EPT = E // NW
K = 80
NCHUNK = EPT // K

@functools.lru_cache(maxsize=1)
def _mesh():
  return plsc.VectorSubcoreMesh(core_axis_name="c", subcore_axis_name="s",
                                num_cores=NC, num_subcores=NS)


def _f32(shape):
  return jax.ShapeDtypeStruct(shape, jnp.float32)


def _i32(shape):
  return jax.ShapeDtypeStruct(shape, jnp.int32)


def _hist_vreg(vals, ref, scr16):
  iota = lax.iota(jnp.int32, 16)
  s, _ = plsc.sort_key_val(vals, vals)
  scr16[...] = s
  prev = plsc.load_gather(scr16, [jnp.maximum(iota - 1, 0)])
  nxt = plsc.load_gather(scr16, [jnp.minimum(iota + 1, 15)])
  run_start = (iota == 0) | (s != prev)
  run_end = (iota == 15) | (s != nxt)
  rsi = plsc.cummax(jnp.where(run_start, iota, 0))
  cnt = (iota - rsi + 1).astype(jnp.float32)
  plsc.addupdate_scatter(ref, [s // 16, s % 16], cnt, mask=run_end)


def _hist_kernel(dst_hbm, batchp_hbm, deg_out, cnt_out,
                 deg_v, cnt_v, chunk_v, bchunk_v, scr16):
  cid = lax.axis_index("c")
  sid = lax.axis_index("s")
  wid = sid * NC + cid

  def z_deg(i, _):
    deg_v[i, :] = jnp.zeros((16,), jnp.float32)
    return 0
  lax.fori_loop(0, 640, z_deg, 0)
  for i in range(16):
    cnt_v[i, :] = jnp.zeros((16,), jnp.float32)

  def chunk_body(c, _):
    pltpu.sync_copy(dst_hbm.at[pl.ds(wid * EPT + c * 400, 400)], chunk_v)
    def vreg_body(j, _):
      _hist_vreg(chunk_v[pl.ds(j * 16, 16)], deg_v, scr16)
      return 0
    lax.fori_loop(0, 25, vreg_body, 0)
    return 0
  lax.fori_loop(0, 25, chunk_body, 0)

  pltpu.sync_copy(batchp_hbm.at[pl.ds(wid * 320, 320)], bchunk_v)
  def bvreg_body(j, _):
    _hist_vreg(bchunk_v[pl.ds(j * 16, 16)], cnt_v, scr16)
    return 0
  lax.fori_loop(0, 20, bvreg_body, 0)

  pltpu.sync_copy(deg_v, deg_out.at[wid])
  pltpu.sync_copy(cnt_v, cnt_out.at[wid])


@functools.lru_cache(maxsize=1)
def _hist():
  @functools.partial(
      pl.kernel,
      out_type=[_f32((NW, 640, 16)), _f32((NW, 16, 16))],
      mesh=_mesh(),
      scratch_types=[
          pltpu.VMEM((640, 16), jnp.float32),
          pltpu.VMEM((16, 16), jnp.float32),
          pltpu.VMEM((400,), jnp.int32),
          pltpu.VMEM((320,), jnp.int32),
          pltpu.VMEM((16,), jnp.int32),
      ],
      compiler_params=pltpu.CompilerParams(needs_layout_passes=False),
  )
  def hist(dst_hbm, batchp_hbm, deg_out, cnt_out, *scratch):
    _hist_kernel(dst_hbm, batchp_hbm, deg_out, cnt_out, *scratch)
  return hist


def _agg_kernel(h_hbm, src_hbm, dst_hbm, p_out,
                r0, r1, r2, r3,
                si0, si1, si2, si3, di0, di1, di2, di3,
                g0, g1, g2, g3, s0, s1, s2, s3, acc):
  cid = lax.axis_index("c")
  sid = lax.axis_index("s")
  wid = sid * NC + cid
  rows = [r0, r1, r2, r3]
  sidxs = [si0, si1, si2, si3]
  didxs = [di0, di1, di2, di3]
  gsems = [g0, g1, g2, g3]
  ssems = [s0, s1, s2, s3]

  def zrow(i, _):
    for j in range(8):
      r0[i, pl.ds(j * 16, 16)] = jnp.zeros((16,), jnp.float32)
    return 0
  lax.fori_loop(0, K, zrow, 0)
  for k in range(8):
    pltpu.sync_copy(r0, acc.at[pl.ds(sid * 640 + k * K, K)])
  plsc.subcore_barrier()

  ebase = wid * EPT
  def fill(c, b):
    pltpu.sync_copy(src_hbm.at[pl.ds(ebase + c * K, K)], sidxs[b])
    pltpu.sync_copy(dst_hbm.at[pl.ds(ebase + c * K, K)], didxs[b])
  def gstart(b):
    pltpu.async_copy(h_hbm.at[sidxs[b]], rows[b], gsems[b])
  def gwait(b):
    pltpu.make_async_copy(h_hbm.at[sidxs[b]], rows[b], gsems[b]).wait()
  def sstart(b):
    pltpu.async_copy(rows[b], acc.at[didxs[b]], ssems[b], add=True)
  def swait(b):
    pltpu.make_async_copy(rows[b], acc.at[didxs[b]], ssems[b]).wait()

  fill(0, 0)
  gstart(0)
  fill(1, 1)
  gstart(1)

  def body(i, _):
    c0 = i * 4
    for b in range(4):
      c = c0 + b
      gwait(b)
      sstart(b)
      @pl.when(c >= 2)
      def _():
        swait((b + 2) % 4)
      @pl.when(c <= NCHUNK - 3)
      def _():
        fill(c + 2, (b + 2) % 4)
        gstart((b + 2) % 4)
    return 0
  lax.fori_loop(0, (NCHUNK - 1) // 4, body, 0)

  gwait(0)
  sstart(0)
  swait(2)
  swait(3)
  swait(0)
  plsc.subcore_barrier()

  for k in range(8):
    rr = sid * 640 + k * K
    pltpu.sync_copy(acc.at[pl.ds(rr, K)], p_out.at[cid, pl.ds(rr, K)])


@functools.lru_cache(maxsize=1)
def _agg():
  @functools.partial(
      pl.kernel,
      out_type=_f32((NC, NP, D)),
      mesh=_mesh(),
      scratch_types=[
          pltpu.VMEM((K, D), jnp.float32),
          pltpu.VMEM((K, D), jnp.float32),
          pltpu.VMEM((K, D), jnp.float32),
          pltpu.VMEM((K, D), jnp.float32),
          pltpu.VMEM((K,), jnp.int32),
          pltpu.VMEM((K,), jnp.int32),
          pltpu.VMEM((K,), jnp.int32),
          pltpu.VMEM((K,), jnp.int32),
          pltpu.VMEM((K,), jnp.int32),
          pltpu.VMEM((K,), jnp.int32),
          pltpu.VMEM((K,), jnp.int32),
          pltpu.VMEM((K,), jnp.int32),
          pltpu.SemaphoreType.DMA,
          pltpu.SemaphoreType.DMA,
          pltpu.SemaphoreType.DMA,
          pltpu.SemaphoreType.DMA,
          pltpu.SemaphoreType.DMA,
          pltpu.SemaphoreType.DMA,
          pltpu.SemaphoreType.DMA,
          pltpu.SemaphoreType.DMA,
          pltpu.VMEM_SHARED((NP, D), jnp.float32),
      ],
      compiler_params=pltpu.CompilerParams(needs_layout_passes=False),
  )
  def agg(h_hbm, src_hbm, dst_hbm, p_out, *scratch):
    _agg_kernel(h_hbm, src_hbm, dst_hbm, p_out, *scratch)
  return agg


def _qgather_kernel(qf_hbm, hs2_hbm, dinv_hbm, gq_hbm, gqb_hbm, b2_hbm,
                    qe_out, dinv_v, b2_v, g8, g8b, dscr,
                    q0v, q1v, hsv, outv):
  cid = lax.axis_index("c")
  sid = lax.axis_index("s")
  wid = sid * NC + cid

  @pl.when(wid < 25)
  def _():
    pltpu.sync_copy(dinv_hbm, dinv_v)
    pltpu.sync_copy(b2_hbm, b2_v)
    pltpu.sync_copy(gq_hbm.at[pl.ds(wid * 8, 8)], g8)
    pltpu.sync_copy(gqb_hbm.at[pl.ds(wid * 8, 8)], g8b)
    iota = lax.iota(jnp.int32, 16)
    g16 = plsc.load_gather(g8, [jnp.minimum(iota, 7)])
    dv16 = plsc.load_gather(dinv_v, [g16])
    dscr[...] = dv16
    pltpu.sync_copy(qf_hbm.at[g8], q0v)
    pltpu.sync_copy(qf_hbm.at[g8b], q1v)
    pltpu.sync_copy(hs2_hbm.at[g8], hsv)
    for r in range(8):
      dr = dv16[r]
      for c in range(8):
        sl = pl.ds(c * 16, 16)
        vec = q0v[r, sl] + q1v[r, sl] + hsv[r, sl]
        outv[r, sl] = dr * vec + b2_v[sl]
    pltpu.sync_copy(outv, qe_out.at[pl.ds(wid * 8, 8)])


@functools.lru_cache(maxsize=1)
def _qgather():
  @functools.partial(
      pl.kernel,
      out_type=_f32((200, D)),
      mesh=_mesh(),
      scratch_types=[
          pltpu.VMEM((NP,), jnp.float32),
          pltpu.VMEM((D,), jnp.float32),
          pltpu.VMEM((8,), jnp.int32),
          pltpu.VMEM((8,), jnp.int32),
          pltpu.VMEM((16,), jnp.float32),
          pltpu.VMEM((8, D), jnp.float32),
          pltpu.VMEM((8, D), jnp.float32),
          pltpu.VMEM((8, D), jnp.float32),
          pltpu.VMEM((8, D), jnp.float32),
      ],
      compiler_params=pltpu.CompilerParams(needs_layout_passes=False),
  )
  def qgather(qf, hs2, dinv, gq, gqb, b2, qe_out, *scratch):
    _qgather_kernel(qf, hs2, dinv, gq, gqb, b2, qe_out, *scratch)
  return qgather


def _t0_body(deg_ref, cnt_ref, targ_ref, dinv_ref, gidx_ref):
  deg = jnp.sum(deg_ref[...], axis=0, keepdims=True)
  dinv_ref[...] = lax.rsqrt(1.0 + deg)
  counts = jnp.sum(cnt_ref[...], axis=0).reshape(1, 256)
  ri = lax.broadcasted_iota(jnp.int32, (256, 128), 0)
  ci = lax.broadcasted_iota(jnp.int32, (256, 128), 1)
  ut = ((ri < ci) & (ri < B)).astype(jnp.float32)
  starts = jnp.dot(counts, ut, preferred_element_type=jnp.float32)
  g = jnp.clip(starts.astype(jnp.int32) + targ_ref[...], 0, N - 1)
  gidx_ref[...] = jnp.concatenate([g, g + NP], axis=0)


def _t0(deg_flat, cnt2, targ2):
  return pl.pallas_call(
      _t0_body,
      out_shape=[_f32((1, NP)), _i32((4, 128))],
  )(deg_flat, cnt2, targ2)


RB = 2000


def _t1_body(x_ref, w_ref, dinv_ref, hs_ref):
  h1p = jnp.dot(x_ref[...], w_ref[...], preferred_element_type=jnp.float32)
  hs_ref[...] = h1p * dinv_ref[...]


def _t1(x, w1, dinv2d):
  return pl.pallas_call(
      _t1_body,
      grid=(N // RB,),
      in_specs=[
          pl.BlockSpec((RB, D), lambda i: (i, 0)),
          pl.BlockSpec((D, D), lambda i: (0, 0)),
          pl.BlockSpec((RB, 1), lambda i: (i, 0)),
      ],
      out_specs=pl.BlockSpec((RB, D), lambda i: (i, 0)),
      out_shape=_f32((N, D)),
  )(x, w1, dinv2d)


def _t2_body(p_ref, hs1_ref, dinv_ref, b1_ref, w2_ref, hs2_ref):
  dinv = dinv_ref[...]
  p = p_ref[...]
  h1 = dinv * (p[0] + p[1] + hs1_ref[...]) + b1_ref[...]
  h1 = jnp.maximum(h1, 0.0)
  h2p = jnp.dot(h1, w2_ref[...], preferred_element_type=jnp.float32)
  hs2_ref[...] = dinv * h2p


def _t2(p, hs1, dinv, b1, w2):
  return pl.pallas_call(
      _t2_body,
      grid=(N // RB,),
      in_specs=[
          pl.BlockSpec((NC, RB, D), lambda i: (0, i, 0)),
          pl.BlockSpec((RB, D), lambda i: (i, 0)),
          pl.BlockSpec((RB, 1), lambda i: (i, 0)),
          pl.BlockSpec((1, D), lambda i: (0, 0)),
          pl.BlockSpec((D, D), lambda i: (0, 0)),
      ],
      out_specs=pl.BlockSpec((RB, D), lambda i: (i, 0)),
      out_shape=_f32((N, D)),
  )(p, hs1, dinv, b1, w2)


def _t4_body(qe_ref, w_ref, b_ref, out_ref):
  out_ref[...] = jnp.dot(qe_ref[...], w_ref[...],
                         preferred_element_type=jnp.float32) + b_ref[...]


def _t4(qe2, w, b):
  return pl.pallas_call(
      _t4_body,
      out_shape=_f32((B, 128)),
  )(qe2, w, b)


def kernel(x, edge_index, batch, target, W1, b1, W2, b2, lin_W, lin_b):
  src = edge_index[0]
  dst = edge_index[1]
  batchp = jnp.concatenate(
      [batch, jnp.full((NW * 320 - N,), B, dtype=jnp.int32)])
  targ2 = jnp.zeros((NC, 128), jnp.int32)
  targ2 = targ2.at[0, :B].set(target[:, 0]).at[1, :B].set(target[:, 1])

  deg_q, cnt_q = _hist()(dst, batchp)
  deg_flat = deg_q.reshape(NW, NP)
  cnt2 = cnt_q.reshape(NW, 256)

  dinv_flat, gidx = _t0(deg_flat, cnt2, targ2)
  dinv2d = dinv_flat.reshape(NP, 1)
  hs1 = _t1(x, W1, dinv2d)
  p = _agg()(hs1, src, dst)
  hs2 = _t2(p, hs1, dinv2d, b1.reshape(1, D), W2)
  q = _agg()(hs2, src, dst)

  gq = jnp.stack([gidx[0, :B], gidx[1, :B]], axis=1).reshape(2 * B)
  gqb = jnp.stack([gidx[2, :B], gidx[3, :B]], axis=1).reshape(2 * B)
  qe = _qgather()(q.reshape(NC * NP, D), hs2, dinv_flat.reshape(NP), gq, gqb, b2)
  qe2 = qe.reshape(B, 2 * D)

  lin_Wp = jnp.zeros((2 * D, 128), jnp.float32).at[:, :18].set(lin_W)
  lin_bp = jnp.zeros((1, 128), jnp.float32).at[0, :18].set(lin_b)
  logits = _t4(qe2, lin_Wp, lin_bp)
  return logits[:, :18]
